# baseline (speedup 1.0000x reference)
import jax
import jax.numpy as jnp
from jax.experimental import pallas as pl
from jax.experimental.pallas import tpu as pltpu

_B = 256
_T = 52
_IN = 52
_NA = 13 * 4 * 13 * 2
_H = _NA * 2
_HAND = 13 * 4 * 13
_HP = 2816
_MB = 64
_NB = _B // _MB

_F32 = jnp.float32
_BF16 = jnp.bfloat16


def _gru_body(n_ref, seq_ref, wih_ref, wrz_ref, wn_ref, bhn_ref,
              h_ref, hb_scr):
    h_ref[...] = jnp.zeros((_B, _H), _F32)
    hb_scr[...] = jnp.zeros((_B, _H), _BF16)

    def step(t, carry):
        n_t = n_ref[t]
        for mb in range(_NB):
            @pl.when(mb * _MB < n_t)
            def _():
                xb = seq_ref[t, mb * _MB:(mb + 1) * _MB, :]
                hb = hb_scr[mb * _MB:(mb + 1) * _MB, :]
                gi = jnp.dot(xb, wih_ref[...],
                             preferred_element_type=_F32)
                prz = jnp.dot(hb, wrz_ref[...],
                              preferred_element_type=_F32)
                ghn = jnp.dot(hb, wn_ref[...],
                              preferred_element_type=_F32)
                r = jax.nn.sigmoid(gi[:, 0:_HP] + prz[:, 0:_HP])
                z = jax.nn.sigmoid(gi[:, _HP:2 * _HP] + prz[:, _HP:2 * _HP])
                n = jnp.tanh(gi[:, 2 * _HP:] + r * (ghn + bhn_ref[...]))
                ho = h_ref[mb * _MB:(mb + 1) * _MB, :]
                n32 = n[:, 0:_H]
                z32 = z[:, 0:_H]
                hnew = n32 + z32 * (ho - n32)
                row = (jax.lax.broadcasted_iota(jnp.int32, (_MB, 1), 0)
                       + mb * _MB)
                hw = jnp.where(row < n_t, hnew, ho)
                h_ref[mb * _MB:(mb + 1) * _MB, :] = hw
                hb_scr[mb * _MB:(mb + 1) * _MB, :] = hw.astype(_BF16)
        return carry

    jax.lax.fori_loop(0, _T, step, 0)


def _head_body(cards_ref, h_ref, mask_ref, w1_ref, b1_ref, w2_ref, b2_ref,
               wdp_ref, bdp_ref, y_ref):
    xh = jnp.maximum(
        jnp.dot(cards_ref[...], w1_ref[...], preferred_element_type=_F32)
        + b1_ref[...], 0.0)
    yh = jnp.dot(xh.astype(_BF16), w2_ref[...],
                 preferred_element_type=_F32) + b2_ref[...]
    xdp = jnp.dot(h_ref[...].astype(_BF16), wdp_ref[...],
                  preferred_element_type=_F32) + bdp_ref[...]
    y_ref[...] = jnp.where(mask_ref[...] > 0, 0.3 * yh + 0.7 * xdp, yh)


def _padn(w, n=_HP):
    return jnp.pad(w, ((0, 0), (0, n - w.shape[1])))


def kernel(cards, discard_pile, hand_fc1_w, hand_fc1_b, hand_fc2_w, hand_fc2_b,
           gru_w_ih, gru_w_hh, gru_b_ih, gru_b_hh, dp_fc1_w, dp_fc1_b):
    slice_sums = discard_pile.reshape(_B, _T, -1).sum(axis=-1)
    zero_slice = slice_sums == 0.0
    has_zero = jnp.any(zero_slice, axis=1)
    lengths = jnp.where(has_zero, jnp.argmax(zero_slice, axis=1), 0)

    perm = jnp.argsort(-lengths)
    inv_perm = jnp.argsort(perm)
    lengths_s = lengths[perm]
    n_arr = jnp.sum(lengths[None, :] > jnp.arange(_T)[:, None],
                    axis=1).astype(jnp.int32)

    seq = jnp.transpose(discard_pile.reshape(_B, _T, _IN)[perm], (1, 0, 2))
    ones = jnp.ones((_T, _B, 1), seq.dtype)
    pad = jnp.zeros((_T, _B, 11), seq.dtype)
    seq = jnp.concatenate([seq, ones, pad], axis=2).astype(_BF16)

    def gi_block(wslice, bias):
        w = jnp.concatenate([wslice.T, bias[None, :],
                             jnp.zeros((11, _H), wslice.dtype)], axis=0)
        return _padn(w)
    wih = jnp.concatenate([
        gi_block(gru_w_ih[:_H], gru_b_ih[:_H] + gru_b_hh[:_H]),
        gi_block(gru_w_ih[_H:2 * _H],
                 gru_b_ih[_H:2 * _H] + gru_b_hh[_H:2 * _H]),
        gi_block(gru_w_ih[2 * _H:], gru_b_ih[2 * _H:]),
    ], axis=1).astype(_BF16)

    wrz = jnp.concatenate([_padn(gru_w_hh[:_H].T),
                           _padn(gru_w_hh[_H:2 * _H].T)],
                          axis=1).astype(_BF16)
    wn = _padn(gru_w_hh[2 * _H:].T).astype(_BF16)
    bhn = _padn(gru_b_hh[2 * _H:][None, :])

    h_s = pl.pallas_call(
        _gru_body,
        out_shape=jax.ShapeDtypeStruct((_B, _H), _F32),
        in_specs=[
            pl.BlockSpec(memory_space=pltpu.SMEM),
            pl.BlockSpec(memory_space=pltpu.VMEM),
            pl.BlockSpec(memory_space=pltpu.VMEM),
            pl.BlockSpec(memory_space=pltpu.VMEM),
            pl.BlockSpec(memory_space=pltpu.VMEM),
            pl.BlockSpec(memory_space=pltpu.VMEM),
        ],
        scratch_shapes=[pltpu.VMEM((_B, _H), _BF16)],
        compiler_params=pltpu.CompilerParams(
            vmem_limit_bytes=64 * 1024 * 1024),
    )(n_arr, seq, wih, wrz, wn, bhn)

    cards_s = cards.reshape(_B, _HAND)[perm].astype(_BF16)
    mask_s = (lengths_s > 0).astype(jnp.int32)[:, None]
    y_s = pl.pallas_call(
        _head_body,
        out_shape=jax.ShapeDtypeStruct((_B, _NA), _F32),
    )(cards_s, h_s, mask_s,
      hand_fc1_w.T.astype(_BF16), hand_fc1_b[None, :],
      hand_fc2_w.T.astype(_BF16), hand_fc2_b[None, :],
      dp_fc1_w.T.astype(_BF16), dp_fc1_b[None, :])
    return y_s[inv_perm]


# speedup vs baseline: 1.3532x; 1.3532x over previous
import jax
import jax.numpy as jnp
from jax.experimental import pallas as pl
from jax.experimental.pallas import tpu as pltpu

_B = 256
_T = 52
_IN = 52
_NA = 13 * 4 * 13 * 2
_H = _NA * 2
_HAND = 13 * 4 * 13
_K = _H + 64

_F32 = jnp.float32
_BF16 = jnp.bfloat16


def _sigmoid(x):
    return 0.5 + 0.5 * jnp.tanh(0.5 * x)


def _gru_body(seq_ref, wrz_ref, wn_ref, win_ref,
              h_ref, mask_ref, hx_scr, valid_scr):
    h_ref[0:_H, :] = jnp.zeros((_H, _B), _F32)
    h_ref[_H:_H + 1, :] = jnp.ones((1, _B), _F32)
    hx_scr[0:_H, :] = jnp.zeros((_H, _B), _BF16)
    valid_scr[...] = jnp.ones((1, _B), _F32)

    def step(t, carry):
        x = seq_ref[t]
        nz = (jnp.sum(x.astype(_F32), axis=0, keepdims=True) - 1.0) != 0.0
        v = jnp.logical_and(valid_scr[...] > 0.0, nz)
        valid_scr[...] = v.astype(_F32)

        @pl.when(t == 0)
        def _():
            mask_ref[...] = nz.astype(jnp.int32)

        hx_scr[_H:_K, :] = x
        hx = hx_scr[...]
        przn = jnp.dot(wrz_ref[...], hx, preferred_element_type=_F32)
        ghn = jnp.dot(wn_ref[...], hx, preferred_element_type=_F32)
        gin = jnp.dot(win_ref[...], hx[_H:_K, :],
                      preferred_element_type=_F32)
        r = _sigmoid(przn[0:_H, :])
        z = _sigmoid(przn[_H:2 * _H, :])
        n = jnp.tanh(gin + r * ghn)
        hs = h_ref[0:_H, :]
        hw = jnp.where(v, n + z * (hs - n), hs)
        h_ref[0:_H, :] = hw
        hx_scr[0:_H, :] = hw.astype(_BF16)
        return carry

    jax.lax.fori_loop(0, _T, step, 0)


def _head_body(cards_ref, h_ref, mask_ref,
               w1_ref, w2_ref, wdp_ref, y_ref, xh_scr):
    xh_scr[0:_H, :] = jnp.maximum(
        jnp.dot(w1_ref[...], cards_ref[...], preferred_element_type=_F32), 0.0)
    xh_scr[_H:_H + 1, :] = jnp.ones((1, _B), _F32)
    yh = jnp.dot(w2_ref[...], xh_scr[...].astype(_BF16),
                 preferred_element_type=_F32)
    xdp = jnp.dot(wdp_ref[...], h_ref[...].astype(_BF16),
                  preferred_element_type=_F32)
    y_ref[...] = jnp.where(mask_ref[...] > 0, 0.3 * yh + 0.7 * xdp, yh)


def _aug(w, b):
    return jnp.concatenate([w, b[:, None]], axis=1).astype(_BF16)


def _kcat(whh, wih, b):
    m = whh.shape[0]
    return jnp.concatenate(
        [whh, wih, b[:, None], jnp.zeros((m, 11), whh.dtype)],
        axis=1).astype(_BF16)


def kernel(cards, discard_pile, hand_fc1_w, hand_fc1_b, hand_fc2_w, hand_fc2_b,
           gru_w_ih, gru_w_hh, gru_b_ih, gru_b_hh, dp_fc1_w, dp_fc1_b):
    seq = jnp.transpose(discard_pile.reshape(_B, _T, _IN), (1, 2, 0))
    seq = jnp.concatenate(
        [seq, jnp.ones((_T, 1, _B), seq.dtype),
         jnp.zeros((_T, 11, _B), seq.dtype)], axis=1).astype(_BF16)

    wrz = jnp.concatenate([
        _kcat(gru_w_hh[:_H], gru_w_ih[:_H],
              gru_b_ih[:_H] + gru_b_hh[:_H]),
        _kcat(gru_w_hh[_H:2 * _H], gru_w_ih[_H:2 * _H],
              gru_b_ih[_H:2 * _H] + gru_b_hh[_H:2 * _H]),
    ], axis=0)
    wn = _kcat(gru_w_hh[2 * _H:], jnp.zeros((_H, _IN), gru_w_hh.dtype),
               gru_b_hh[2 * _H:])
    win = jnp.concatenate(
        [gru_w_ih[2 * _H:], gru_b_ih[2 * _H:][:, None],
         jnp.zeros((_H, 11), gru_w_ih.dtype)], axis=1).astype(_BF16)

    h_aug, mask = pl.pallas_call(
        _gru_body,
        out_shape=[
            jax.ShapeDtypeStruct((_H + 1, _B), _F32),
            jax.ShapeDtypeStruct((1, _B), jnp.int32),
        ],
        scratch_shapes=[pltpu.VMEM((_K, _B), _BF16),
                        pltpu.VMEM((1, _B), _F32)],
        compiler_params=pltpu.CompilerParams(
            vmem_limit_bytes=64 * 1024 * 1024),
    )(seq, wrz, wn, win)

    cards_t = jnp.concatenate(
        [cards.reshape(_B, _HAND).T, jnp.ones((1, _B), cards.dtype)],
        axis=0).astype(_BF16)
    w1 = _aug(hand_fc1_w, hand_fc1_b)
    w2 = _aug(hand_fc2_w, hand_fc2_b)
    wdp = _aug(dp_fc1_w, dp_fc1_b)

    y_t = pl.pallas_call(
        _head_body,
        out_shape=jax.ShapeDtypeStruct((_NA, _B), _F32),
        scratch_shapes=[pltpu.VMEM((_H + 1, _B), _F32)],
    )(cards_t, h_aug, mask, w1, w2, wdp)
    return y_t.T


# speedup vs baseline: 1.3993x; 1.0341x over previous
import jax
import jax.numpy as jnp
from jax.experimental import pallas as pl
from jax.experimental.pallas import tpu as pltpu

_B = 256
_T = 52
_IN = 52
_NA = 13 * 4 * 13 * 2
_H = _NA * 2
_HAND = 13 * 4 * 13

_F32 = jnp.float32
_BF16 = jnp.bfloat16


def _sigmoid(x):
    return 0.5 + 0.5 * jnp.tanh(0.5 * x)


def _gru_body(seq_ref, wih_r_ref, wih_z_ref, wih_n_ref,
              whh_r_ref, whh_z_ref, whh_n_ref,
              h_ref, mask_ref, valid_scr):
    h_ref[0:_H, :] = jnp.zeros((_H, _B), _F32)
    h_ref[_H:_H + 1, :] = jnp.ones((1, _B), _F32)
    valid_scr[...] = jnp.ones((1, _B), _F32)

    def step(t, carry):
        x = seq_ref[t]
        nz = (jnp.sum(x.astype(_F32), axis=0, keepdims=True) - 1.0) != 0.0
        v = jnp.logical_and(valid_scr[...] > 0.0, nz)
        valid_scr[...] = v.astype(_F32)

        @pl.when(t == 0)
        def _():
            mask_ref[...] = nz.astype(jnp.int32)

        h = h_ref[...]
        hb = h.astype(_BF16)
        hs_b = hb[:_H, :]
        r = _sigmoid(
            jnp.dot(wih_r_ref[...], x, preferred_element_type=_F32)
            + jnp.dot(whh_r_ref[...], hs_b, preferred_element_type=_F32))
        n = jnp.tanh(
            jnp.dot(wih_n_ref[...], x, preferred_element_type=_F32)
            + r * jnp.dot(whh_n_ref[...], hb, preferred_element_type=_F32))
        z = _sigmoid(
            jnp.dot(wih_z_ref[...], x, preferred_element_type=_F32)
            + jnp.dot(whh_z_ref[...], hs_b, preferred_element_type=_F32))
        hs = h[:_H, :]
        h_ref[0:_H, :] = jnp.where(v, n + z * (hs - n), hs)
        return carry

    jax.lax.fori_loop(0, _T, step, 0)


def _head_body(cards_ref, h_ref, mask_ref,
               w1_ref, w2_ref, wdp_ref, y_ref, xh_scr):
    xh_scr[0:_H, :] = jnp.maximum(
        jnp.dot(w1_ref[...], cards_ref[...], preferred_element_type=_F32), 0.0)
    xh_scr[_H:_H + 1, :] = jnp.ones((1, _B), _F32)
    yh = jnp.dot(w2_ref[...], xh_scr[...].astype(_BF16),
                 preferred_element_type=_F32)
    xdp = jnp.dot(wdp_ref[...], h_ref[...].astype(_BF16),
                  preferred_element_type=_F32)
    y_ref[...] = jnp.where(mask_ref[...] > 0, 0.3 * yh + 0.7 * xdp, yh)


def _aug(w, b):
    return jnp.concatenate([w, b[:, None]], axis=1).astype(_BF16)


def kernel(cards, discard_pile, hand_fc1_w, hand_fc1_b, hand_fc2_w, hand_fc2_b,
           gru_w_ih, gru_w_hh, gru_b_ih, gru_b_hh, dp_fc1_w, dp_fc1_b):
    seq = jnp.transpose(discard_pile.reshape(_B, _T, _IN), (1, 2, 0))
    seq = jnp.concatenate(
        [seq, jnp.ones((_T, 1, _B), seq.dtype)], axis=1).astype(_BF16)

    wih_r = _aug(gru_w_ih[:_H], gru_b_ih[:_H] + gru_b_hh[:_H])
    wih_z = _aug(gru_w_ih[_H:2 * _H],
                 gru_b_ih[_H:2 * _H] + gru_b_hh[_H:2 * _H])
    wih_n = _aug(gru_w_ih[2 * _H:], gru_b_ih[2 * _H:])
    whh_r = gru_w_hh[:_H].astype(_BF16)
    whh_z = gru_w_hh[_H:2 * _H].astype(_BF16)
    whh_n = _aug(gru_w_hh[2 * _H:], gru_b_hh[2 * _H:])

    h_aug, mask = pl.pallas_call(
        _gru_body,
        out_shape=[
            jax.ShapeDtypeStruct((_H + 1, _B), _F32),
            jax.ShapeDtypeStruct((1, _B), jnp.int32),
        ],
        scratch_shapes=[pltpu.VMEM((1, _B), _F32)],
        compiler_params=pltpu.CompilerParams(
            vmem_limit_bytes=64 * 1024 * 1024),
    )(seq, wih_r, wih_z, wih_n, whh_r, whh_z, whh_n)

    cards_t = jnp.concatenate(
        [cards.reshape(_B, _HAND).T, jnp.ones((1, _B), cards.dtype)],
        axis=0).astype(_BF16)
    w1 = _aug(hand_fc1_w, hand_fc1_b)
    w2 = _aug(hand_fc2_w, hand_fc2_b)
    wdp = _aug(dp_fc1_w, dp_fc1_b)

    y_t = pl.pallas_call(
        _head_body,
        out_shape=jax.ShapeDtypeStruct((_NA, _B), _F32),
        scratch_shapes=[pltpu.VMEM((_H + 1, _B), _F32)],
    )(cards_t, h_aug, mask, w1, w2, wdp)
    return y_t.T


# speedup vs baseline: 1.4065x; 1.0052x over previous
import jax
import jax.numpy as jnp
from jax.experimental import pallas as pl
from jax.experimental.pallas import tpu as pltpu

_B = 256
_T = 52
_IN = 52
_NA = 13 * 4 * 13 * 2
_H = _NA * 2
_HAND = 13 * 4 * 13

_F32 = jnp.float32
_BF16 = jnp.bfloat16


def _gru_body(seq_ref, wih_r_ref, wih_z_ref, wih_n_ref,
              whh_r_ref, whh_z_ref, whh_n_ref,
              h_ref, mask_ref, valid_scr):
    h_ref[0:_H, :] = jnp.zeros((_H, _B), _F32)
    h_ref[_H:_H + 1, :] = jnp.ones((1, _B), _F32)
    valid_scr[...] = jnp.ones((1, _B), _F32)

    def step(t, carry):
        x = seq_ref[t]
        nz = (jnp.sum(x.astype(_F32), axis=0, keepdims=True) - 1.0) != 0.0
        v = jnp.logical_and(valid_scr[...] > 0.0, nz)
        valid_scr[...] = v.astype(_F32)

        @pl.when(t == 0)
        def _():
            mask_ref[...] = nz.astype(jnp.int32)

        h = h_ref[...]
        hb = h.astype(_BF16)
        hs_b = hb[:_H, :]
        r = jax.nn.sigmoid(
            jnp.dot(wih_r_ref[...], x, preferred_element_type=_F32)
            + jnp.dot(whh_r_ref[...], hs_b, preferred_element_type=_F32))
        n = jnp.tanh(
            jnp.dot(wih_n_ref[...], x, preferred_element_type=_F32)
            + r * jnp.dot(whh_n_ref[...], hb, preferred_element_type=_F32))
        z = jax.nn.sigmoid(
            jnp.dot(wih_z_ref[...], x, preferred_element_type=_F32)
            + jnp.dot(whh_z_ref[...], hs_b, preferred_element_type=_F32))
        hs = h[:_H, :]
        h_ref[0:_H, :] = jnp.where(v, n + z * (hs - n), hs)
        return carry

    jax.lax.fori_loop(0, _T, step, 0)


def _head_body(cards_ref, h_ref, mask_ref,
               w1_ref, w2_ref, wdp_ref, y_ref, xh_scr):
    xh_scr[0:_H, :] = jnp.maximum(
        jnp.dot(w1_ref[...], cards_ref[...], preferred_element_type=_F32), 0.0)
    xh_scr[_H:_H + 1, :] = jnp.ones((1, _B), _F32)
    yh = jnp.dot(w2_ref[...], xh_scr[...].astype(_BF16),
                 preferred_element_type=_F32)
    xdp = jnp.dot(wdp_ref[...], h_ref[...].astype(_BF16),
                  preferred_element_type=_F32)
    y_ref[...] = jnp.where(mask_ref[...] > 0, 0.3 * yh + 0.7 * xdp, yh)


def _aug(w, b):
    return jnp.concatenate([w, b[:, None]], axis=1).astype(_BF16)


def kernel(cards, discard_pile, hand_fc1_w, hand_fc1_b, hand_fc2_w, hand_fc2_b,
           gru_w_ih, gru_w_hh, gru_b_ih, gru_b_hh, dp_fc1_w, dp_fc1_b):
    seq = jnp.transpose(discard_pile.reshape(_B, _T, _IN), (1, 2, 0))
    seq = jnp.concatenate(
        [seq, jnp.ones((_T, 1, _B), seq.dtype)], axis=1).astype(_BF16)

    wih_r = _aug(gru_w_ih[:_H], gru_b_ih[:_H] + gru_b_hh[:_H])
    wih_z = _aug(gru_w_ih[_H:2 * _H],
                 gru_b_ih[_H:2 * _H] + gru_b_hh[_H:2 * _H])
    wih_n = _aug(gru_w_ih[2 * _H:], gru_b_ih[2 * _H:])
    whh_r = gru_w_hh[:_H].astype(_BF16)
    whh_z = gru_w_hh[_H:2 * _H].astype(_BF16)
    whh_n = _aug(gru_w_hh[2 * _H:], gru_b_hh[2 * _H:])

    h_aug, mask = pl.pallas_call(
        _gru_body,
        out_shape=[
            jax.ShapeDtypeStruct((_H + 1, _B), _F32),
            jax.ShapeDtypeStruct((1, _B), jnp.int32),
        ],
        scratch_shapes=[pltpu.VMEM((1, _B), _F32)],
        compiler_params=pltpu.CompilerParams(
            vmem_limit_bytes=64 * 1024 * 1024),
    )(seq, wih_r, wih_z, wih_n, whh_r, whh_z, whh_n)

    cards_t = jnp.concatenate(
        [cards.reshape(_B, _HAND).T, jnp.ones((1, _B), cards.dtype)],
        axis=0).astype(_BF16)
    w1 = _aug(hand_fc1_w, hand_fc1_b)
    w2 = _aug(hand_fc2_w, hand_fc2_b)
    wdp = _aug(dp_fc1_w, dp_fc1_b)

    y_t = pl.pallas_call(
        _head_body,
        out_shape=jax.ShapeDtypeStruct((_NA, _B), _F32),
        scratch_shapes=[pltpu.VMEM((_H + 1, _B), _F32)],
    )(cards_t, h_aug, mask, w1, w2, wdp)
    return y_t.T
